# baseline (speedup 1.0000x reference)
import functools

import jax
import jax.numpy as jnp
from jax import lax
from jax.experimental import pallas as pl
from jax.experimental.pallas import tpu as pltpu
from jax.experimental.pallas import tpu_sc as plsc

L = 16
CHUNK = 128


@functools.cache
def _build(B, D, NC, NS):
    NW = NC * NS
    b_per_w = B // NW
    n_chunks = b_per_w // CHUNK
    mesh = plsc.VectorSubcoreMesh(core_axis_name="c", subcore_axis_name="s")

    @functools.partial(
        pl.kernel,
        mesh=mesh,
        compiler_params=pltpu.CompilerParams(
            needs_layout_passes=False, use_tc_tiling_on_sc=False),
        out_type=(
            jax.ShapeDtypeStruct((NW, b_per_w), jnp.float32),
            jax.ShapeDtypeStruct((NW, b_per_w), jnp.float32),
        ),
        scratch_types=[
            pltpu.VMEM((n_chunks, CHUNK), jnp.int32),
            pltpu.VMEM((n_chunks, CHUNK), jnp.int32),
            pltpu.VMEM((n_chunks, CHUNK), jnp.int32),
            pltpu.VMEM((b_per_w, D), jnp.float32),
            pltpu.VMEM((b_per_w, D), jnp.float32),
            pltpu.VMEM((b_per_w, D), jnp.float32),
            pltpu.VMEM((b_per_w,), jnp.float32),
            pltpu.VMEM((b_per_w,), jnp.float32),
            pltpu.SemaphoreType.DMA,
        ],
    )
    def bpr_kernel(uid_hbm, pid_hbm, nid_hbm, utab, atab, pos_out, neg_out,
                   uidx, pidx, nidx, urows, prows, nrows, posv, negv, sem):
        wid = lax.axis_index("s") * NC + lax.axis_index("c")
        pltpu.sync_copy(uid_hbm.at[wid], uidx)
        pltpu.sync_copy(pid_hbm.at[wid], pidx)
        pltpu.sync_copy(nid_hbm.at[wid], nidx)
        copies = []
        for j in range(n_chunks):
            dst = pl.ds(j * CHUNK, CHUNK)
            copies.append(pltpu.async_copy(utab.at[uidx.at[j]], urows.at[dst], sem))
            copies.append(pltpu.async_copy(atab.at[pidx.at[j]], prows.at[dst], sem))
            copies.append(pltpu.async_copy(atab.at[nidx.at[j]], nrows.at[dst], sem))
        for c in copies:
            c.wait()

        lane = lax.iota(jnp.int32, L)

        def body(g, _):
            rows = lane + g * L
            pacc = jnp.zeros((L,), jnp.float32)
            nacc = jnp.zeros((L,), jnp.float32)
            for d in range(D):
                col = jnp.full((L,), d, jnp.int32)
                u = plsc.load_gather(urows, [rows, col])
                p = plsc.load_gather(prows, [rows, col])
                nn = plsc.load_gather(nrows, [rows, col])
                pacc = pacc + u * p
                nacc = nacc + u * nn
            posv[pl.ds(g * L, L)] = pacc
            negv[pl.ds(g * L, L)] = nacc
            return _

        lax.fori_loop(0, b_per_w // L, body, None)
        pltpu.sync_copy(posv, pos_out.at[wid])
        pltpu.sync_copy(negv, neg_out.at[wid])

    return bpr_kernel


def kernel(user_ids, pos_action_ids, neg_action_ids, user_table, action_table):
    B = user_ids.shape[0]
    D = user_table.shape[1]
    info = plsc.get_sparse_core_info()
    NC, NS = info.num_cores, info.num_subcores
    NW = NC * NS
    b_per_w = B // NW
    n_chunks = b_per_w // CHUNK
    uid = user_ids.astype(jnp.int32).reshape(NW, n_chunks, CHUNK)
    pid = pos_action_ids.astype(jnp.int32).reshape(NW, n_chunks, CHUNK)
    nid = neg_action_ids.astype(jnp.int32).reshape(NW, n_chunks, CHUNK)
    pos, neg = _build(B, D, NC, NS)(uid, pid, nid, user_table, action_table)
    return pos.reshape(B), neg.reshape(B)
